# baseline (speedup 1.0000x reference)
import functools

import jax
import jax.numpy as jnp
import numpy as np
from jax.experimental import pallas as pl
from jax.experimental.pallas import tpu as pltpu
from jax.experimental.pallas import tpu_sc as plsc

EMBED = 256
NH = 8
NL = 4
NP = 8
NC = 6
DZ = 4
HD = 32
NQ = 2500
NQP = 2560
LP = NL * NP
LANES = NH * LP

_SP = np.array([[46, 80], [23, 40], [12, 20], [6, 10]], dtype=np.int64)
_SIZES = (_SP[:, 0] * _SP[:, 1]).astype(np.int64)
_LSTART = np.concatenate([[0], np.cumsum(_SIZES)[:-1]])
_PL = ((_SP[:, 0] + 1) * (_SP[:, 1] + 1)).astype(np.int64)
_PBASE = np.concatenate([[0], np.cumsum(_PL)[:-1]])
PTOT = int(_PL.sum())
TAB_ROWS = NC * NH * PTOT
S_TOTAL = NC * NQP * LANES
GWIN = 128

_j = np.arange(LANES)
_lh = _j // LP
_ll = (_j % LP) // NP
LANE_W = _SP[_ll, 1].astype(np.float32).reshape(1, LANES)
LANE_H = _SP[_ll, 0].astype(np.float32).reshape(1, LANES)
LANE_WP1 = (_SP[_ll, 1] + 1).astype(np.int32).reshape(1, LANES)
LANE_BASE = (_lh * PTOT + _PBASE[_ll]).astype(np.int32).reshape(1, LANES)


def _prep0_body(q_ref, qp_ref, wo_ref, bo_ref, wa_ref, ba_ref, off_ref, aw_ref):
    qs = q_ref[...] + qp_ref[...]
    off_ref[...] = jnp.dot(qs, wo_ref[...], preferred_element_type=jnp.float32) + bo_ref[...]
    a = jnp.dot(qs, wa_ref[...], preferred_element_type=jnp.float32) + ba_ref[...]
    a3 = a.reshape(a.shape[0], NH, LP)
    m = jnp.max(a3, axis=-1, keepdims=True)
    e = jnp.exp(a3 - m)
    sm = e / jnp.sum(e, axis=-1, keepdims=True)
    aw_ref[...] = sm.reshape(a.shape)


def _prep0(qpad, qpospad, W_off, b_off, W_attn, b_attn):
    blk = 512
    return pl.pallas_call(
        _prep0_body,
        grid=(NQP // blk,),
        in_specs=[pl.BlockSpec((blk, EMBED), lambda i: (i, 0)),
                  pl.BlockSpec((blk, EMBED), lambda i: (i, 0)),
                  pl.BlockSpec((EMBED, 512), lambda i: (0, 0)),
                  pl.BlockSpec((1, 512), lambda i: (0, 0)),
                  pl.BlockSpec((EMBED, EMBED), lambda i: (0, 0)),
                  pl.BlockSpec((1, EMBED), lambda i: (0, 0))],
        out_specs=[pl.BlockSpec((blk, 512), lambda i: (i, 0)),
                   pl.BlockSpec((blk, EMBED), lambda i: (i, 0))],
        out_shape=[jax.ShapeDtypeStruct((NQP, 512), jnp.float32),
                   jax.ShapeDtypeStruct((NQP, EMBED), jnp.float32)],
    )(qpad, qpospad, W_off, b_off.reshape(1, 512), W_attn, b_attn.reshape(1, EMBED))


def _maskscale_body(bev_ref, scale_ref):
    m = (jnp.sum(bev_ref[...], axis=-1) > 0).astype(jnp.float32)
    cnt = jnp.clip(jnp.sum(m, axis=0, keepdims=True), 1.0, None)
    scale_ref[...] = m / cnt


def _maskscale(bevf):
    return pl.pallas_call(
        _maskscale_body,
        out_shape=jax.ShapeDtypeStruct((NC, NQP), jnp.float32),
    )(bevf)


def _valproj_body(v_ref, w_ref, b_ref, o_ref):
    o_ref[0] = (jnp.dot(v_ref[0], w_ref[...], preferred_element_type=jnp.float32)
                + b_ref[...]).astype(jnp.bfloat16)


def _valproj(value6, W_v, b_v):
    L = value6.shape[1]
    return pl.pallas_call(
        _valproj_body,
        grid=(NC,),
        in_specs=[pl.BlockSpec((1, L, EMBED), lambda c: (c, 0, 0)),
                  pl.BlockSpec((EMBED, EMBED), lambda c: (0, 0)),
                  pl.BlockSpec((1, EMBED), lambda c: (0, 0))],
        out_specs=pl.BlockSpec((1, L, EMBED), lambda c: (c, 0, 0)),
        out_shape=jax.ShapeDtypeStruct((NC, L, EMBED), jnp.bfloat16),
    )(value6, W_v, b_v.reshape(1, EMBED))


def _prep1_body(offx_ref, offy_ref, refx_ref, refy_ref, awm_ref, scale_ref,
                lw_ref, lh_ref, lwp1_ref, lbase_ref, idx_ref,
                w00_ref, w01_ref, w10_ref, w11_ref):
    c = pl.program_id(0)
    lw = lw_ref[...]
    lh = lh_ref[...]
    locx = refx_ref[0] + offx_ref[...] / lw
    locy = refy_ref[0] + offy_ref[...] / lh
    px = locx * lw - 0.5
    py = locy * lh - 0.5
    x0 = jnp.floor(px)
    y0 = jnp.floor(py)
    fx = px - x0
    fy = py - y0
    vx0 = ((x0 >= 0) & (x0 <= lw - 1)).astype(jnp.float32)
    vx1 = ((x0 + 1 >= 0) & (x0 + 1 <= lw - 1)).astype(jnp.float32)
    vy0 = ((y0 >= 0) & (y0 <= lh - 1)).astype(jnp.float32)
    vy1 = ((y0 + 1 >= 0) & (y0 + 1 <= lh - 1)).astype(jnp.float32)
    xc = jnp.clip(x0, -1.0, lw - 1).astype(jnp.int32)
    yc = jnp.clip(y0, -1.0, lh - 1).astype(jnp.int32)
    row = (yc + 1) * lwp1_ref[...] + (xc + 1)
    idx_ref[0] = row + lbase_ref[...] + c * (NH * PTOT)
    bw = awm_ref[...] * scale_ref[0]
    wx0 = 1.0 - fx
    wy0 = 1.0 - fy
    w00_ref[0] = bw * (wy0 * wx0 * vy0 * vx0)
    w01_ref[0] = bw * (wy0 * fx * vy0 * vx1)
    w10_ref[0] = bw * (fy * wx0 * vy1 * vx0)
    w11_ref[0] = bw * (fy * fx * vy1 * vx1)


def _prep1(offx, offy, refx, refy, awm, scale3):
    blk = 512
    f = jnp.float32
    return pl.pallas_call(
        _prep1_body,
        grid=(NC, NQP // blk),
        in_specs=[pl.BlockSpec((blk, LANES), lambda c, i: (i, 0)),
                  pl.BlockSpec((blk, LANES), lambda c, i: (i, 0)),
                  pl.BlockSpec((1, blk, LANES), lambda c, i: (c, i, 0)),
                  pl.BlockSpec((1, blk, LANES), lambda c, i: (c, i, 0)),
                  pl.BlockSpec((blk, LANES), lambda c, i: (i, 0)),
                  pl.BlockSpec((1, blk, 1), lambda c, i: (c, i, 0)),
                  pl.BlockSpec((1, LANES), lambda c, i: (0, 0)),
                  pl.BlockSpec((1, LANES), lambda c, i: (0, 0)),
                  pl.BlockSpec((1, LANES), lambda c, i: (0, 0)),
                  pl.BlockSpec((1, LANES), lambda c, i: (0, 0))],
        out_specs=[pl.BlockSpec((1, blk, LANES), lambda c, i: (c, i, 0))] * 5,
        out_shape=[jax.ShapeDtypeStruct((NC, NQP, LANES), jnp.int32)]
        + [jax.ShapeDtypeStruct((NC, NQP, LANES), jnp.float32)] * 4,
    )(offx, offy, refx, refy, awm, scale3,
      jnp.asarray(LANE_W), jnp.asarray(LANE_H),
      jnp.asarray(LANE_WP1), jnp.asarray(LANE_BASE))


def _sc_gather(tab, idx2d):
    mesh = plsc.VectorSubcoreMesh(core_axis_name="c", subcore_axis_name="s")

    @functools.partial(
        pl.kernel,
        out_type=jax.ShapeDtypeStruct((S_TOTAL, 64), jnp.int32),
        mesh=mesh,
        compiler_params=pltpu.CompilerParams(use_tc_tiling_on_sc=False),
    )
    def sck(tab_hbm, idx_hbm, g_hbm):
        def body(i_vmem, o_vmem):
            pltpu.sync_copy(tab_hbm.at[i_vmem.at[0]], o_vmem)

        pltpu.emit_pipeline(
            body,
            grid=(S_TOTAL // GWIN,),
            in_specs=[pl.BlockSpec((1, GWIN), index_map=lambda i: (0, i))],
            out_specs=[pl.BlockSpec((GWIN, 64), index_map=lambda i: (i, 0))],
            core_axis_name=("c", "s"),
            dimension_semantics=(pltpu.PARALLEL,),
        )(idx_hbm, g_hbm)

    return sck(tab, idx2d)


QB = 16

def _blend_body(g_ref, w_ref, e_ref, o_ref):
    c = pl.program_id(1)
    gi = g_ref[0].reshape(QB * LANES, 64)
    ge = jax.lax.bitcast_convert_type(jax.lax.shift_left(gi, 16), jnp.float32)
    go = jax.lax.bitcast_convert_type(gi & jnp.int32(-65536), jnp.float32)
    w = w_ref[0]
    wexp = jnp.dot(w, e_ref[...], preferred_element_type=jnp.float32)
    pe = (ge * wexp).reshape(QB * NH, LP, 64)
    po = (go * wexp).reshape(QB * NH, LP, 64)
    ye = pe[:, 0:8] + pe[:, 8:16] + pe[:, 16:24] + pe[:, 24:32]
    yo = po[:, 0:8] + po[:, 8:16] + po[:, 16:24] + po[:, 24:32]
    se = jnp.sum(ye, axis=1)
    so = jnp.sum(yo, axis=1)
    oe = se[:, 0:16] + se[:, 16:32] + se[:, 32:48] + se[:, 48:64]
    oo = so[:, 0:16] + so[:, 16:32] + so[:, 32:48] + so[:, 48:64]
    blk = jnp.concatenate([oe, oo], axis=1)

    @pl.when(c == 0)
    def _():
        o_ref[...] = blk

    @pl.when(c != 0)
    def _():
        o_ref[...] += blk


def _blend(G4, W4r):
    emat = jnp.asarray(np.repeat(np.eye(4, dtype=np.float32), 16, axis=1))
    return pl.pallas_call(
        _blend_body,
        grid=(NQP // QB, NC),
        in_specs=[pl.BlockSpec((1, QB, LANES, 64), lambda i, c: (c, i, 0, 0)),
                  pl.BlockSpec((1, QB * LANES, 4), lambda i, c: (c, i, 0)),
                  pl.BlockSpec((4, 64), lambda i, c: (0, 0))],
        out_specs=pl.BlockSpec((QB * NH, HD), lambda i, c: (i, 0)),
        out_shape=jax.ShapeDtypeStruct((NQP * NH, HD), jnp.float32),
    )(G4, W4r, emat)


def _outproj_body(x_ref, w_ref, b_ref, r_ref, o_ref):
    o_ref[...] = (jnp.dot(x_ref[...], w_ref[...], preferred_element_type=jnp.float32)
                  + b_ref[...] + r_ref[...])


def _outproj(x, W, b, resid):
    return pl.pallas_call(
        _outproj_body,
        out_shape=jax.ShapeDtypeStruct((NQ, EMBED), jnp.float32),
    )(x, W, b.reshape(1, EMBED), resid)


def _build_patch_table(vp):
    pats = []
    for lvl in range(NL):
        h, w = int(_SP[lvl, 0]), int(_SP[lvl, 1])
        s = int(_LSTART[lvl])
        seg = vp[:, s:s + h * w].reshape(NC, h, w, NH, HD)
        seg = seg.transpose(0, 3, 1, 2, 4)
        seg = jnp.pad(seg, ((0, 0), (0, 0), (1, 1), (1, 1), (0, 0)))
        a = seg[:, :, 0:h + 1, 0:w + 1]
        b = seg[:, :, 0:h + 1, 1:w + 2]
        cc = seg[:, :, 1:h + 2, 0:w + 1]
        d = seg[:, :, 1:h + 2, 1:w + 2]
        pat = jnp.concatenate([a, b, cc, d], axis=-1)
        pats.append(pat.reshape(NC, NH, int(_PL[lvl]), 128))
    tab = jnp.concatenate(pats, axis=2)
    return tab.reshape(TAB_ROWS, 128)


def kernel(query, key, value, query_pos, reference_points_cam, bev_mask,
           spatial_shapes, level_start_index, W_v, b_v, W_off, b_off,
           W_attn, b_attn, W_out, b_out):
    f = jnp.float32
    qpad = jnp.pad(query[0], ((0, NQP - NQ), (0, 0)))
    qpospad = jnp.pad(query_pos[0], ((0, NQP - NQ), (0, 0)))

    off_lin, awm = _prep0(qpad, qpospad, W_off, b_off, W_attn, b_attn)
    offx = off_lin.reshape(NQP, LANES, 2)[..., 0]
    offy = off_lin.reshape(NQP, LANES, 2)[..., 1]

    ref6 = reference_points_cam[:, 0]
    refx = jnp.pad(jnp.tile(ref6[..., 0], (1, 1, LANES // DZ)),
                   ((0, 0), (0, NQP - NQ), (0, 0)))
    refy = jnp.pad(jnp.tile(ref6[..., 1], (1, 1, LANES // DZ)),
                   ((0, 0), (0, NQP - NQ), (0, 0)))

    bevf = jnp.pad(bev_mask[:, 0].astype(f), ((0, 0), (0, NQP - NQ), (0, 0)))
    scale = _maskscale(bevf)
    scale3 = scale.reshape(NC, NQP, 1)

    idx4, w00, w01, w10, w11 = _prep1(offx, offy, refx, refy, awm, scale3)
    W4r = jnp.stack([w00, w01, w10, w11], axis=-1).reshape(NC, NQP * LANES, 4)

    vp = _valproj(value[:, :, 0, :], W_v, b_v)
    tab_b = _build_patch_table(vp)
    tab = jax.lax.bitcast_convert_type(tab_b.reshape(TAB_ROWS, 64, 2), jnp.int32)

    G = _sc_gather(tab, idx4.reshape(1, S_TOTAL))
    G4 = G.reshape(NC, NQP, LANES, 64)

    outq = _blend(G4, W4r).reshape(NQP, EMBED)
    perm = np.concatenate([h * HD + np.r_[np.arange(0, HD, 2), np.arange(1, HD, 2)]
                           for h in range(NH)]).astype(np.int32)
    res = _outproj(outq[:NQ], W_out[jnp.asarray(perm), :], b_out, query[0])
    return res[None]


# speedup vs baseline: 2.4266x; 2.4266x over previous
import functools

import jax
import jax.numpy as jnp
import numpy as np
from jax.experimental import pallas as pl
from jax.experimental.pallas import tpu as pltpu
from jax.experimental.pallas import tpu_sc as plsc

EMBED = 256
NH = 8
NL = 4
NP = 8
NC = 6
DZ = 4
HD = 32
NQ = 2500
NQP = 2560
LP = NL * NP
LANES = NH * LP

_SP = np.array([[46, 80], [23, 40], [12, 20], [6, 10]], dtype=np.int64)
_SIZES = (_SP[:, 0] * _SP[:, 1]).astype(np.int64)
_LSTART = np.concatenate([[0], np.cumsum(_SIZES)[:-1]])
_PL = ((_SP[:, 0] + 1) * (_SP[:, 1] + 1)).astype(np.int64)
_PBASE = np.concatenate([[0], np.cumsum(_PL)[:-1]])
PTOT = int(_PL.sum())
TAB_ROWS = NC * NH * PTOT
S_TOTAL = NC * NQP * LANES
GWIN = 128

_j = np.arange(LANES)
_lh = _j // LP
_ll = (_j % LP) // NP
LANE_W = _SP[_ll, 1].astype(np.float32).reshape(1, LANES)
LANE_H = _SP[_ll, 0].astype(np.float32).reshape(1, LANES)
LANE_WP1 = (_SP[_ll, 1] + 1).astype(np.int32).reshape(1, LANES)
LANE_BASE = (_lh * PTOT + _PBASE[_ll]).astype(np.int32).reshape(1, LANES)


def _prep0_body(q_ref, qp_ref, wo_ref, bo_ref, wa_ref, ba_ref, off_ref, aw_ref):
    qs = q_ref[...] + qp_ref[...]
    off_ref[...] = jnp.dot(qs, wo_ref[...], preferred_element_type=jnp.float32) + bo_ref[...]
    a = jnp.dot(qs, wa_ref[...], preferred_element_type=jnp.float32) + ba_ref[...]
    a3 = a.reshape(a.shape[0], NH, LP)
    m = jnp.max(a3, axis=-1, keepdims=True)
    e = jnp.exp(a3 - m)
    sm = e / jnp.sum(e, axis=-1, keepdims=True)
    aw_ref[...] = sm.reshape(a.shape)


def _prep0(qpad, qpospad, W_off, b_off, W_attn, b_attn):
    blk = 512
    return pl.pallas_call(
        _prep0_body,
        grid=(NQP // blk,),
        in_specs=[pl.BlockSpec((blk, EMBED), lambda i: (i, 0)),
                  pl.BlockSpec((blk, EMBED), lambda i: (i, 0)),
                  pl.BlockSpec((EMBED, 512), lambda i: (0, 0)),
                  pl.BlockSpec((1, 512), lambda i: (0, 0)),
                  pl.BlockSpec((EMBED, EMBED), lambda i: (0, 0)),
                  pl.BlockSpec((1, EMBED), lambda i: (0, 0))],
        out_specs=[pl.BlockSpec((blk, 512), lambda i: (i, 0)),
                   pl.BlockSpec((blk, EMBED), lambda i: (i, 0))],
        out_shape=[jax.ShapeDtypeStruct((NQP, 512), jnp.float32),
                   jax.ShapeDtypeStruct((NQP, EMBED), jnp.float32)],
    )(qpad, qpospad, W_off, b_off.reshape(1, 512), W_attn, b_attn.reshape(1, EMBED))


def _maskscale_body(bev_ref, scale_ref):
    m = (jnp.sum(bev_ref[...], axis=-1) > 0).astype(jnp.float32)
    cnt = jnp.clip(jnp.sum(m, axis=0, keepdims=True), 1.0, None)
    scale_ref[...] = m / cnt


def _maskscale(bevf):
    return pl.pallas_call(
        _maskscale_body,
        out_shape=jax.ShapeDtypeStruct((NC, NQP), jnp.float32),
    )(bevf)


def _valproj_body(v_ref, w_ref, b_ref, o_ref):
    o_ref[0] = jnp.dot(v_ref[0], w_ref[...], preferred_element_type=jnp.float32) + b_ref[...]


def _valproj(value6, W_v, b_v):
    L = value6.shape[1]
    return pl.pallas_call(
        _valproj_body,
        grid=(NC,),
        in_specs=[pl.BlockSpec((1, L, EMBED), lambda c: (c, 0, 0)),
                  pl.BlockSpec((EMBED, EMBED), lambda c: (0, 0)),
                  pl.BlockSpec((1, EMBED), lambda c: (0, 0))],
        out_specs=pl.BlockSpec((1, L, EMBED), lambda c: (c, 0, 0)),
        out_shape=jax.ShapeDtypeStruct((NC, L, EMBED), jnp.float32),
    )(value6, W_v, b_v.reshape(1, EMBED))


def _prep1_body(offx_ref, offy_ref, refx_ref, refy_ref, awm_ref, scale_ref,
                lw_ref, lh_ref, lwp1_ref, lbase_ref, idx_ref,
                w00_ref, w01_ref, w10_ref, w11_ref):
    c = pl.program_id(0)
    lw = lw_ref[...]
    lh = lh_ref[...]
    locx = refx_ref[0] + offx_ref[...] / lw
    locy = refy_ref[0] + offy_ref[...] / lh
    px = locx * lw - 0.5
    py = locy * lh - 0.5
    x0 = jnp.floor(px)
    y0 = jnp.floor(py)
    fx = px - x0
    fy = py - y0
    vx0 = ((x0 >= 0) & (x0 <= lw - 1)).astype(jnp.float32)
    vx1 = ((x0 + 1 >= 0) & (x0 + 1 <= lw - 1)).astype(jnp.float32)
    vy0 = ((y0 >= 0) & (y0 <= lh - 1)).astype(jnp.float32)
    vy1 = ((y0 + 1 >= 0) & (y0 + 1 <= lh - 1)).astype(jnp.float32)
    xc = jnp.clip(x0, -1.0, lw - 1).astype(jnp.int32)
    yc = jnp.clip(y0, -1.0, lh - 1).astype(jnp.int32)
    row = (yc + 1) * lwp1_ref[...] + (xc + 1)
    idx_ref[0] = row + lbase_ref[...] + c * (NH * PTOT)
    bw = awm_ref[...] * scale_ref[0]
    wx0 = 1.0 - fx
    wy0 = 1.0 - fy
    w00_ref[0] = bw * (wy0 * wx0 * vy0 * vx0)
    w01_ref[0] = bw * (wy0 * fx * vy0 * vx1)
    w10_ref[0] = bw * (fy * wx0 * vy1 * vx0)
    w11_ref[0] = bw * (fy * fx * vy1 * vx1)


def _prep1(offx, offy, refx, refy, awm, scale3):
    blk = 512
    f = jnp.float32
    return pl.pallas_call(
        _prep1_body,
        grid=(NC, NQP // blk),
        in_specs=[pl.BlockSpec((blk, LANES), lambda c, i: (i, 0)),
                  pl.BlockSpec((blk, LANES), lambda c, i: (i, 0)),
                  pl.BlockSpec((1, blk, LANES), lambda c, i: (c, i, 0)),
                  pl.BlockSpec((1, blk, LANES), lambda c, i: (c, i, 0)),
                  pl.BlockSpec((blk, LANES), lambda c, i: (i, 0)),
                  pl.BlockSpec((1, blk, 1), lambda c, i: (c, i, 0)),
                  pl.BlockSpec((1, LANES), lambda c, i: (0, 0)),
                  pl.BlockSpec((1, LANES), lambda c, i: (0, 0)),
                  pl.BlockSpec((1, LANES), lambda c, i: (0, 0)),
                  pl.BlockSpec((1, LANES), lambda c, i: (0, 0))],
        out_specs=[pl.BlockSpec((1, blk, LANES), lambda c, i: (c, i, 0))] * 5,
        out_shape=[jax.ShapeDtypeStruct((NC, NQP, LANES), jnp.int32)]
        + [jax.ShapeDtypeStruct((NC, NQP, LANES), jnp.float32)] * 4,
    )(offx, offy, refx, refy, awm, scale3,
      jnp.asarray(LANE_W), jnp.asarray(LANE_H),
      jnp.asarray(LANE_WP1), jnp.asarray(LANE_BASE))


def _sc_gather(tab, idx2d):
    mesh = plsc.VectorSubcoreMesh(core_axis_name="c", subcore_axis_name="s")

    @functools.partial(
        pl.kernel,
        out_type=jax.ShapeDtypeStruct((S_TOTAL, 128), jnp.float32),
        mesh=mesh,
    )
    def sck(tab_hbm, idx_hbm, g_hbm):
        def body(i_vmem, o_vmem):
            pltpu.sync_copy(tab_hbm.at[i_vmem.at[0]], o_vmem)

        pltpu.emit_pipeline(
            body,
            grid=(S_TOTAL // GWIN,),
            in_specs=[pl.BlockSpec((1, GWIN), index_map=lambda i: (0, i))],
            out_specs=[pl.BlockSpec((GWIN, 128), index_map=lambda i: (i, 0))],
            core_axis_name=("c", "s"),
            dimension_semantics=(pltpu.PARALLEL,),
        )(idx_hbm, g_hbm)

    return sck(tab, idx2d)


QB = 16

def _blend_body(g_ref, w_ref, e_ref, o_ref):
    c = pl.program_id(1)
    g = g_ref[0].reshape(QB * LANES, 128)
    w = w_ref[0]
    wexp = jnp.dot(w, e_ref[...], preferred_element_type=jnp.float32)
    p = (g * wexp).reshape(QB * NH, LP, 128)
    y = p[:, 0:8] + p[:, 8:16] + p[:, 16:24] + p[:, 24:32]
    s = jnp.sum(y, axis=1)
    blk = s[:, 0:32] + s[:, 32:64] + s[:, 64:96] + s[:, 96:128]

    @pl.when(c == 0)
    def _():
        o_ref[...] = blk

    @pl.when(c != 0)
    def _():
        o_ref[...] += blk


def _blend(G4, W4r):
    emat = jnp.asarray(np.repeat(np.eye(4, dtype=np.float32), HD, axis=1))
    return pl.pallas_call(
        _blend_body,
        grid=(NQP // QB, NC),
        in_specs=[pl.BlockSpec((1, QB, LANES, 128), lambda i, c: (c, i, 0, 0)),
                  pl.BlockSpec((1, QB * LANES, 4), lambda i, c: (c, i, 0)),
                  pl.BlockSpec((4, 128), lambda i, c: (0, 0))],
        out_specs=pl.BlockSpec((QB * NH, HD), lambda i, c: (i, 0)),
        out_shape=jax.ShapeDtypeStruct((NQP * NH, HD), jnp.float32),
    )(G4, W4r, emat)


def _outproj_body(x_ref, w_ref, b_ref, r_ref, o_ref):
    o_ref[...] = (jnp.dot(x_ref[...], w_ref[...], preferred_element_type=jnp.float32)
                  + b_ref[...] + r_ref[...])


def _outproj(x, W, b, resid):
    return pl.pallas_call(
        _outproj_body,
        out_shape=jax.ShapeDtypeStruct((NQ, EMBED), jnp.float32),
    )(x, W, b.reshape(1, EMBED), resid)


def _build_patch_table(vp):
    pats = []
    for lvl in range(NL):
        h, w = int(_SP[lvl, 0]), int(_SP[lvl, 1])
        s = int(_LSTART[lvl])
        seg = vp[:, s:s + h * w].reshape(NC, h, w, NH, HD)
        seg = seg.transpose(0, 3, 1, 2, 4)
        seg = jnp.pad(seg, ((0, 0), (0, 0), (1, 1), (1, 1), (0, 0)))
        a = seg[:, :, 0:h + 1, 0:w + 1]
        b = seg[:, :, 0:h + 1, 1:w + 2]
        cc = seg[:, :, 1:h + 2, 0:w + 1]
        d = seg[:, :, 1:h + 2, 1:w + 2]
        pat = jnp.concatenate([a, b, cc, d], axis=-1)
        pats.append(pat.reshape(NC, NH, int(_PL[lvl]), 128))
    tab = jnp.concatenate(pats, axis=2)
    return tab.reshape(TAB_ROWS, 128)


def kernel(query, key, value, query_pos, reference_points_cam, bev_mask,
           spatial_shapes, level_start_index, W_v, b_v, W_off, b_off,
           W_attn, b_attn, W_out, b_out):
    f = jnp.float32
    qpad = jnp.pad(query[0], ((0, NQP - NQ), (0, 0)))
    qpospad = jnp.pad(query_pos[0], ((0, NQP - NQ), (0, 0)))

    off_lin, awm = _prep0(qpad, qpospad, W_off, b_off, W_attn, b_attn)
    offx = off_lin.reshape(NQP, LANES, 2)[..., 0]
    offy = off_lin.reshape(NQP, LANES, 2)[..., 1]

    ref6 = reference_points_cam[:, 0]
    refx = jnp.pad(jnp.tile(ref6[..., 0], (1, 1, LANES // DZ)),
                   ((0, 0), (0, NQP - NQ), (0, 0)))
    refy = jnp.pad(jnp.tile(ref6[..., 1], (1, 1, LANES // DZ)),
                   ((0, 0), (0, NQP - NQ), (0, 0)))

    bevf = jnp.pad(bev_mask[:, 0].astype(f), ((0, 0), (0, NQP - NQ), (0, 0)))
    scale = _maskscale(bevf)
    scale3 = scale.reshape(NC, NQP, 1)

    idx4, w00, w01, w10, w11 = _prep1(offx, offy, refx, refy, awm, scale3)
    W4r = jnp.stack([w00, w01, w10, w11], axis=-1).reshape(NC, NQP * LANES, 4)

    vp = _valproj(value[:, :, 0, :], W_v, b_v)
    tab = _build_patch_table(vp)

    G = _sc_gather(tab, idx4.reshape(1, S_TOTAL))
    G4 = G.reshape(NC, NQP, LANES, 128)

    outq = _blend(G4, W4r).reshape(NQP, EMBED)
    res = _outproj(outq[:NQ], W_out, b_out, query[0])
    return res[None]


# speedup vs baseline: 7.4509x; 3.0704x over previous
import functools

import jax
import jax.numpy as jnp
import numpy as np
from jax.experimental import pallas as pl
from jax.experimental.pallas import tpu as pltpu
from jax.experimental.pallas import tpu_sc as plsc

EMBED = 256
NH = 8
NL = 4
NP = 8
NC = 6
DZ = 4
HD = 32
NQ = 2500
NQP = 2560
LP = NL * NP
LANES = NH * LP

_SP = np.array([[46, 80], [23, 40], [12, 20], [6, 10]], dtype=np.int64)
_SIZES = (_SP[:, 0] * _SP[:, 1]).astype(np.int64)
_LSTART = np.concatenate([[0], np.cumsum(_SIZES)[:-1]])
_PL = ((_SP[:, 0] + 1) * (_SP[:, 1] + 1)).astype(np.int64)
_PBASE = np.concatenate([[0], np.cumsum(_PL)[:-1]])
PTOT = int(_PL.sum())
TAB_ROWS = NC * NH * PTOT
S_TOTAL = NC * NQP * LANES
GWIN = 128

_j = np.arange(LANES)
_lh = _j // LP
_ll = (_j % LP) // NP
LANE_W = _SP[_ll, 1].astype(np.float32).reshape(1, LANES)
LANE_H = _SP[_ll, 0].astype(np.float32).reshape(1, LANES)
LANE_WP1 = (_SP[_ll, 1] + 1).astype(np.int32).reshape(1, LANES)
LANE_BASE = (_lh * PTOT + _PBASE[_ll]).astype(np.int32).reshape(1, LANES)


def _prep0_body(q_ref, qp_ref, wo_ref, bo_ref, wa_ref, ba_ref, off_ref, aw_ref):
    qs = q_ref[...] + qp_ref[...]
    off_ref[...] = jnp.dot(qs, wo_ref[...], preferred_element_type=jnp.float32) + bo_ref[...]
    a = jnp.dot(qs, wa_ref[...], preferred_element_type=jnp.float32) + ba_ref[...]
    a3 = a.reshape(a.shape[0], NH, LP)
    m = jnp.max(a3, axis=-1, keepdims=True)
    e = jnp.exp(a3 - m)
    sm = e / jnp.sum(e, axis=-1, keepdims=True)
    aw_ref[...] = sm.reshape(a.shape)


def _prep0(qpad, qpospad, W_off, b_off, W_attn, b_attn):
    blk = 512
    return pl.pallas_call(
        _prep0_body,
        grid=(NQP // blk,),
        in_specs=[pl.BlockSpec((blk, EMBED), lambda i: (i, 0)),
                  pl.BlockSpec((blk, EMBED), lambda i: (i, 0)),
                  pl.BlockSpec((EMBED, 512), lambda i: (0, 0)),
                  pl.BlockSpec((1, 512), lambda i: (0, 0)),
                  pl.BlockSpec((EMBED, EMBED), lambda i: (0, 0)),
                  pl.BlockSpec((1, EMBED), lambda i: (0, 0))],
        out_specs=[pl.BlockSpec((blk, 512), lambda i: (i, 0)),
                   pl.BlockSpec((blk, EMBED), lambda i: (i, 0))],
        out_shape=[jax.ShapeDtypeStruct((NQP, 512), jnp.float32),
                   jax.ShapeDtypeStruct((NQP, EMBED), jnp.float32)],
    )(qpad, qpospad, W_off, b_off.reshape(1, 512), W_attn, b_attn.reshape(1, EMBED))


def _maskscale_body(bev_ref, scale_ref):
    m = (jnp.sum(bev_ref[...], axis=-1) > 0).astype(jnp.float32)
    cnt = jnp.clip(jnp.sum(m, axis=0, keepdims=True), 1.0, None)
    scale_ref[...] = m / cnt


def _maskscale(bevf):
    return pl.pallas_call(
        _maskscale_body,
        out_shape=jax.ShapeDtypeStruct((NC, NQP), jnp.float32),
    )(bevf)


def _valproj_body(v_ref, w_ref, b_ref, o_ref):
    o_ref[0] = jnp.dot(v_ref[0], w_ref[...], preferred_element_type=jnp.float32) + b_ref[...]


def _valproj(value6, W_v, b_v):
    L = value6.shape[1]
    return pl.pallas_call(
        _valproj_body,
        grid=(NC,),
        in_specs=[pl.BlockSpec((1, L, EMBED), lambda c: (c, 0, 0)),
                  pl.BlockSpec((EMBED, EMBED), lambda c: (0, 0)),
                  pl.BlockSpec((1, EMBED), lambda c: (0, 0))],
        out_specs=pl.BlockSpec((1, L, EMBED), lambda c: (c, 0, 0)),
        out_shape=jax.ShapeDtypeStruct((NC, L, EMBED), jnp.float32),
    )(value6, W_v, b_v.reshape(1, EMBED))


def _prep1_body(offx_ref, offy_ref, refx_ref, refy_ref, awm_ref, scale_ref,
                lw_ref, lh_ref, lwp1_ref, lbase_ref, idx_ref,
                w00_ref, w01_ref, w10_ref, w11_ref):
    c = pl.program_id(0)
    lw = lw_ref[...]
    lh = lh_ref[...]
    locx = refx_ref[0] + offx_ref[...] / lw
    locy = refy_ref[0] + offy_ref[...] / lh
    px = locx * lw - 0.5
    py = locy * lh - 0.5
    x0 = jnp.floor(px)
    y0 = jnp.floor(py)
    fx = px - x0
    fy = py - y0
    vx0 = ((x0 >= 0) & (x0 <= lw - 1)).astype(jnp.float32)
    vx1 = ((x0 + 1 >= 0) & (x0 + 1 <= lw - 1)).astype(jnp.float32)
    vy0 = ((y0 >= 0) & (y0 <= lh - 1)).astype(jnp.float32)
    vy1 = ((y0 + 1 >= 0) & (y0 + 1 <= lh - 1)).astype(jnp.float32)
    xc = jnp.clip(x0, -1.0, lw - 1).astype(jnp.int32)
    yc = jnp.clip(y0, -1.0, lh - 1).astype(jnp.int32)
    row = (yc + 1) * lwp1_ref[...] + (xc + 1)
    idx_ref[0] = row + lbase_ref[...] + c * (NH * PTOT)
    bw = awm_ref[...] * scale_ref[0]
    wx0 = 1.0 - fx
    wy0 = 1.0 - fy
    w00_ref[0] = bw * (wy0 * wx0 * vy0 * vx0)
    w01_ref[0] = bw * (wy0 * fx * vy0 * vx1)
    w10_ref[0] = bw * (fy * wx0 * vy1 * vx0)
    w11_ref[0] = bw * (fy * fx * vy1 * vx1)


def _prep1(offx, offy, refx, refy, awm, scale3):
    blk = 512
    f = jnp.float32
    return pl.pallas_call(
        _prep1_body,
        grid=(NC, NQP // blk),
        in_specs=[pl.BlockSpec((blk, LANES), lambda c, i: (i, 0)),
                  pl.BlockSpec((blk, LANES), lambda c, i: (i, 0)),
                  pl.BlockSpec((1, blk, LANES), lambda c, i: (c, i, 0)),
                  pl.BlockSpec((1, blk, LANES), lambda c, i: (c, i, 0)),
                  pl.BlockSpec((blk, LANES), lambda c, i: (i, 0)),
                  pl.BlockSpec((1, blk, 1), lambda c, i: (c, i, 0)),
                  pl.BlockSpec((1, LANES), lambda c, i: (0, 0)),
                  pl.BlockSpec((1, LANES), lambda c, i: (0, 0)),
                  pl.BlockSpec((1, LANES), lambda c, i: (0, 0)),
                  pl.BlockSpec((1, LANES), lambda c, i: (0, 0))],
        out_specs=[pl.BlockSpec((1, blk, LANES), lambda c, i: (c, i, 0))] * 5,
        out_shape=[jax.ShapeDtypeStruct((NC, NQP, LANES), jnp.int32)]
        + [jax.ShapeDtypeStruct((NC, NQP, LANES), jnp.float32)] * 4,
    )(offx, offy, refx, refy, awm, scale3,
      jnp.asarray(LANE_W), jnp.asarray(LANE_H),
      jnp.asarray(LANE_WP1), jnp.asarray(LANE_BASE))


def _sc_gather(tab, idx2d):
    mesh = plsc.VectorSubcoreMesh(core_axis_name="c", subcore_axis_name="s")

    @functools.partial(
        pl.kernel,
        out_type=jax.ShapeDtypeStruct((S_TOTAL, 128), jnp.float32),
        mesh=mesh,
    )
    def sck(tab_hbm, idx_hbm, g_hbm):
        def body(i_vmem, o_vmem):
            pltpu.sync_copy(tab_hbm.at[i_vmem.at[0]], o_vmem)

        pltpu.emit_pipeline(
            body,
            grid=(S_TOTAL // GWIN,),
            in_specs=[pl.BlockSpec((1, GWIN), index_map=lambda i: (0, i))],
            out_specs=[pl.BlockSpec((GWIN, 128), index_map=lambda i: (i, 0))],
            core_axis_name=("c", "s"),
            dimension_semantics=(pltpu.PARALLEL,),
        )(idx_hbm, g_hbm)

    return sck(tab, idx2d)


QB = 16

def _blend_body(g_ref, w_ref, e_ref, o_ref):
    c = pl.program_id(1)
    g = g_ref[0].reshape(QB * LANES, 128)
    w = w_ref[0]
    wexp = jax.lax.dot_general(w, e_ref[...], (((0,), (0,)), ((), ())),
                               preferred_element_type=jnp.float32)
    p = (g * wexp).reshape(QB * NH, LP, 128)
    y = p[:, 0:8] + p[:, 8:16] + p[:, 16:24] + p[:, 24:32]
    s = jnp.sum(y, axis=1)
    blk = s[:, 0:32] + s[:, 32:64] + s[:, 64:96] + s[:, 96:128]

    @pl.when(c == 0)
    def _():
        o_ref[...] = blk

    @pl.when(c != 0)
    def _():
        o_ref[...] += blk


def _blend(G4, W4r):
    emat = jnp.asarray(np.repeat(np.eye(4, dtype=np.float32), HD, axis=1))
    return pl.pallas_call(
        _blend_body,
        grid=(NQP // QB, NC),
        in_specs=[pl.BlockSpec((1, QB, LANES, 128), lambda i, c: (c, i, 0, 0)),
                  pl.BlockSpec((1, 4, QB * LANES), lambda i, c: (c, 0, i)),
                  pl.BlockSpec((4, 128), lambda i, c: (0, 0))],
        out_specs=pl.BlockSpec((QB * NH, HD), lambda i, c: (i, 0)),
        out_shape=jax.ShapeDtypeStruct((NQP * NH, HD), jnp.float32),
    )(G4, W4r, emat)


def _outproj_body(x_ref, w_ref, b_ref, r_ref, o_ref):
    o_ref[...] = (jnp.dot(x_ref[...], w_ref[...], preferred_element_type=jnp.float32)
                  + b_ref[...] + r_ref[...])


def _outproj(x, W, b, resid):
    return pl.pallas_call(
        _outproj_body,
        out_shape=jax.ShapeDtypeStruct((NQ, EMBED), jnp.float32),
    )(x, W, b.reshape(1, EMBED), resid)


def _build_patch_table(vp):
    pats = []
    for lvl in range(NL):
        h, w = int(_SP[lvl, 0]), int(_SP[lvl, 1])
        s = int(_LSTART[lvl])
        seg = vp[:, s:s + h * w].reshape(NC, h, w, NH, HD)
        seg = seg.transpose(0, 3, 1, 2, 4)
        seg = jnp.pad(seg, ((0, 0), (0, 0), (1, 1), (1, 1), (0, 0)))
        a = seg[:, :, 0:h + 1, 0:w + 1]
        b = seg[:, :, 0:h + 1, 1:w + 2]
        cc = seg[:, :, 1:h + 2, 0:w + 1]
        d = seg[:, :, 1:h + 2, 1:w + 2]
        pat = jnp.concatenate([a, b, cc, d], axis=-1)
        pats.append(pat.reshape(NC, NH, int(_PL[lvl]), 128))
    tab = jnp.concatenate(pats, axis=2)
    return tab.reshape(TAB_ROWS, 128)


def kernel(query, key, value, query_pos, reference_points_cam, bev_mask,
           spatial_shapes, level_start_index, W_v, b_v, W_off, b_off,
           W_attn, b_attn, W_out, b_out):
    f = jnp.float32
    qpad = jnp.pad(query[0], ((0, NQP - NQ), (0, 0)))
    qpospad = jnp.pad(query_pos[0], ((0, NQP - NQ), (0, 0)))

    off_lin, awm = _prep0(qpad, qpospad, W_off, b_off, W_attn, b_attn)
    offx = off_lin.reshape(NQP, LANES, 2)[..., 0]
    offy = off_lin.reshape(NQP, LANES, 2)[..., 1]

    ref6 = reference_points_cam[:, 0]
    refx = jnp.pad(jnp.tile(ref6[..., 0], (1, 1, LANES // DZ)),
                   ((0, 0), (0, NQP - NQ), (0, 0)))
    refy = jnp.pad(jnp.tile(ref6[..., 1], (1, 1, LANES // DZ)),
                   ((0, 0), (0, NQP - NQ), (0, 0)))

    bevf = jnp.pad(bev_mask[:, 0].astype(f), ((0, 0), (0, NQP - NQ), (0, 0)))
    scale = _maskscale(bevf)
    scale3 = scale.reshape(NC, NQP, 1)

    idx4, w00, w01, w10, w11 = _prep1(offx, offy, refx, refy, awm, scale3)
    W4r = jnp.stack([w00, w01, w10, w11], axis=1).reshape(NC, 4, NQP * LANES)

    vp = _valproj(value[:, :, 0, :], W_v, b_v)
    tab = _build_patch_table(vp)

    G = _sc_gather(tab, idx4.reshape(1, S_TOTAL))
    G4 = G.reshape(NC, NQP, LANES, 128)

    outq = _blend(G4, W4r).reshape(NQP, EMBED)
    res = _outproj(outq[:NQ], W_out, b_out, query[0])
    return res[None]
